# baseline (speedup 1.0000x reference)
import functools

import jax
import jax.numpy as jnp
from jax import lax
from jax.experimental import pallas as pl
from jax.experimental.pallas import tpu as pltpu
from jax.experimental.pallas import tpu_sc as plsc

_B = 16384
_S = 50
_F = 32
_NC, _NS = 2, 16
_NW = _NC * _NS
_QB = 1024
_NQ = _B // _QB
_NJOB = _S * _NQ
_PER_W = _NJOB // _NW
_GROUP = 128

_mesh = plsc.VectorSubcoreMesh(core_axis_name="c", subcore_axis_name="s")


@functools.partial(
    pl.kernel,
    mesh=_mesh,
    out_type=jax.ShapeDtypeStruct((_S, _F // 8, _B // 128, 8, 128),
                                  jnp.float32),
    scratch_types=[
        pltpu.VMEM((_QB,), jnp.int32),
        pltpu.VMEM((_QB,), jnp.int32),
        pltpu.VMEM((_QB, _F), jnp.float32),
        pltpu.VMEM((_QB, _F), jnp.float32),
        pltpu.VMEM((_F // 8, _QB // 128, 8, 128), jnp.float32),
        pltpu.SemaphoreType.DMA,
        pltpu.SemaphoreType.DMA,
        pltpu.SemaphoreType.DMA,
    ],
    compiler_params=pltpu.CompilerParams(
        use_tc_tiling_on_sc=False, needs_layout_passes=False),
)
def _embed_gather(idx_hbm, table_hbm, out_hbm, idx0, idx1, rows0, rows1,
                  outt_v, semg0, semg1, sems):
    wid = lax.axis_index("s") * _NC + lax.axis_index("c")
    jbase = wid * _PER_W

    def fetch(t, idx_v, rows_v, semg):
        jg = jbase + t
        s = jg // _NQ
        q = jg % _NQ
        pltpu.sync_copy(idx_hbm.at[s, pl.ds(q * _QB, _QB)], idx_v)
        for r in range(_QB // _GROUP):
            pltpu.async_copy(
                table_hbm.at[idx_v.at[pl.ds(r * _GROUP, _GROUP)]],
                rows_v.at[pl.ds(r * _GROUP, _GROUP), :],
                semg,
            )

    def wait_rows(rows_v, semg):
        pltpu.make_async_copy(
            table_hbm.at[pl.ds(0, _QB)], rows_v, semg).wait()

    def drain_store(first):
        @pl.when(jnp.logical_not(first))
        def _():
            pltpu.make_async_copy(
                outt_v, out_hbm.at[0, :, pl.ds(0, _QB // 128)], sems).wait()

    def transpose_store(t, rows_v, first):
        jg = jbase + t
        s = jg // _NQ
        q = jg % _NQ

        drain_store(first)

        @plsc.parallel_loop(0, _QB // 16, unroll=4)
        def tgroup(g):
            r0 = g * 16
            bc = g // 8
            row_ids = r0 + lax.iota(jnp.int32, 16)
            vals = [
                plsc.load_gather(
                    rows_v, [row_ids, jnp.full((16,), f, jnp.int32)])
                for f in range(_F)
            ]
            for f in range(_F):
                outt_v[f // 8, bc, f % 8, pl.ds((g % 8) * 16, 16)] = vals[f]

        for fb in range(_F // 8):
            pltpu.async_copy(
                outt_v.at[fb],
                out_hbm.at[s, fb, pl.ds(q * (_QB // 128), _QB // 128)],
                sems,
            )

    fetch(0, idx0, rows0, semg0)

    def pair(i, carry):
        ta = 2 * i
        fetch(ta + 1, idx1, rows1, semg1)
        wait_rows(rows0, semg0)
        transpose_store(ta, rows0, i == 0)

        @pl.when(i < _PER_W // 2 - 1)
        def _():
            fetch(ta + 2, idx0, rows0, semg0)

        wait_rows(rows1, semg1)
        transpose_store(ta + 1, rows1, False)
        return carry

    lax.fori_loop(0, _PER_W // 2, pair, 0)

    fetch(_PER_W - 1, idx0, rows0, semg0)
    wait_rows(rows0, semg0)
    transpose_store(_PER_W - 1, rows0, False)
    drain_store(False)


def kernel(inputs, embedding):
    out5 = _embed_gather(inputs.T, embedding)
    return out5.transpose((2, 4, 0, 1, 3)).reshape(_B, _S, _F)


# speedup vs baseline: 1.1145x; 1.1145x over previous
import functools

import jax
import jax.numpy as jnp
from jax import lax
from jax.experimental import pallas as pl
from jax.experimental.pallas import tpu as pltpu
from jax.experimental.pallas import tpu_sc as plsc

_B = 16384
_S = 50
_F = 32
_NC, _NS = 2, 16
_NW = _NC * _NS
_QB = 1024
_NQ = _B // _QB
_NJOB = _S * _NQ
_PER_W = _NJOB // _NW
_GROUP = 128

_mesh = plsc.VectorSubcoreMesh(core_axis_name="c", subcore_axis_name="s")


@functools.partial(
    pl.kernel,
    mesh=_mesh,
    out_type=jax.ShapeDtypeStruct((_S, _F // 8, _B // 128, 8, 128),
                                  jnp.float32),
    scratch_types=[
        pltpu.VMEM((_QB,), jnp.int32),
        pltpu.VMEM((_QB,), jnp.int32),
        pltpu.VMEM((_QB, _F), jnp.float32),
        pltpu.VMEM((_QB, _F), jnp.float32),
        pltpu.VMEM((_F // 8, _QB // 128, 8, 128), jnp.float32),
        pltpu.SemaphoreType.DMA,
        pltpu.SemaphoreType.DMA,
        pltpu.SemaphoreType.DMA,
    ],
    compiler_params=pltpu.CompilerParams(
        use_tc_tiling_on_sc=False, needs_layout_passes=False),
)
def _embed_gather(idx_hbm, table_hbm, out_hbm, idx0, idx1, rows0, rows1,
                  outt_v, semg0, semg1, sems):
    wid = lax.axis_index("s") * _NC + lax.axis_index("c")
    jbase = wid * _PER_W

    def fetch(t, idx_v, rows_v, semg):
        jg = jbase + t
        s = jg // _NQ
        q = jg % _NQ
        pltpu.sync_copy(idx_hbm.at[s, pl.ds(q * _QB, _QB)], idx_v)
        for r in range(_QB // _GROUP):
            pltpu.async_copy(
                table_hbm.at[idx_v.at[pl.ds(r * _GROUP, _GROUP)]],
                rows_v.at[pl.ds(r * _GROUP, _GROUP), :],
                semg,
            )

    def wait_rows(rows_v, semg):
        pltpu.make_async_copy(
            table_hbm.at[pl.ds(0, _QB)], rows_v, semg).wait()

    def drain_store(first):
        @pl.when(jnp.logical_not(first))
        def _():
            pltpu.make_async_copy(
                outt_v, out_hbm.at[0, :, pl.ds(0, _QB // 128)], sems).wait()

    def transpose_store(t, rows_v, first):
        jg = jbase + t
        s = jg // _NQ
        q = jg % _NQ

        drain_store(first)

        f16 = lax.iota(jnp.int32, 16)
        fb0 = f16 >> 3
        fr0 = f16 & 7

        @plsc.parallel_loop(0, _QB, unroll=4)
        def trow(r):
            bcv = jnp.full((16,), r >> 7, jnp.int32)
            bwv = jnp.full((16,), r & 127, jnp.int32)
            v0 = rows_v[r, pl.ds(0, 16)]
            v1 = rows_v[r, pl.ds(16, 16)]
            plsc.store_scatter(outt_v, [fb0, bcv, fr0, bwv], v0)
            plsc.store_scatter(outt_v, [fb0 + 2, bcv, fr0, bwv], v1)

        for fb in range(_F // 8):
            pltpu.async_copy(
                outt_v.at[fb],
                out_hbm.at[s, fb, pl.ds(q * (_QB // 128), _QB // 128)],
                sems,
            )

    fetch(0, idx0, rows0, semg0)

    def pair(i, carry):
        ta = 2 * i
        fetch(ta + 1, idx1, rows1, semg1)
        wait_rows(rows0, semg0)
        transpose_store(ta, rows0, i == 0)

        @pl.when(i < _PER_W // 2 - 1)
        def _():
            fetch(ta + 2, idx0, rows0, semg0)

        wait_rows(rows1, semg1)
        transpose_store(ta + 1, rows1, False)
        return carry

    lax.fori_loop(0, _PER_W // 2, pair, 0)

    fetch(_PER_W - 1, idx0, rows0, semg0)
    wait_rows(rows0, semg0)
    transpose_store(_PER_W - 1, rows0, False)
    drain_store(False)


def kernel(inputs, embedding):
    out5 = _embed_gather(inputs.T, embedding)
    return out5.transpose((2, 4, 0, 1, 3)).reshape(_B, _S, _F)
